# baseline (speedup 1.0000x reference)
import functools

import jax
import jax.numpy as jnp
from jax import lax
from jax.experimental import pallas as pl
from jax.experimental.pallas import tpu as pltpu
from jax.experimental.pallas import tpu_sc as plsc

MAX_LEN = 128
SIDE = MAX_LEN + 1
FLAT = SIDE * SIDE
FLAT_PAD = 16704

_info = plsc.get_sparse_core_info()
NC = _info.num_cores
NS = _info.num_subcores
L = _info.num_lanes
NW = NC * NS


def _make_lookup(B: int):
    b_per_w = B // NW
    n_vec = b_per_w // L
    mesh = plsc.VectorSubcoreMesh(core_axis_name="c", subcore_axis_name="s")

    @functools.partial(
        pl.kernel,
        mesh=mesh,
        out_type=jax.ShapeDtypeStruct((B,), jnp.float32),
        compiler_params=pltpu.CompilerParams(needs_layout_passes=False),
        scratch_types=[
            pltpu.VMEM((FLAT_PAD,), jnp.float32),
            pltpu.VMEM((b_per_w,), jnp.int32),
            pltpu.VMEM((b_per_w,), jnp.int32),
            pltpu.VMEM((b_per_w,), jnp.float32),
        ],
    )
    def lookup(var_hbm, pre_hbm, tab_hbm, out_hbm, tab_v, var_v, pre_v, out_v):
        wid = lax.axis_index("s") * NC + lax.axis_index("c")
        base = wid * b_per_w
        pltpu.sync_copy(tab_hbm, tab_v)
        pltpu.sync_copy(var_hbm.at[pl.ds(base, b_per_w)], var_v)
        pltpu.sync_copy(pre_hbm.at[pl.ds(base, b_per_w)], pre_v)

        for i in range(n_vec):
            off = i * L
            v = var_v[pl.ds(off, L)]
            p = pre_v[pl.ds(off, L)]
            flat = v * SIDE + p
            out_v[pl.ds(off, L)] = plsc.load_gather(tab_v, [flat])
        pltpu.sync_copy(out_v, out_hbm.at[pl.ds(base, b_per_w)])

    return lookup


def kernel(var_len, prefix_idx, table):
    B = var_len.shape[0]
    tflat = jnp.pad(table.reshape(-1), (0, FLAT_PAD - FLAT))
    fn = _make_lookup(B)
    return fn(
        var_len.astype(jnp.int32),
        prefix_idx.astype(jnp.int32),
        tflat.astype(jnp.float32),
    )


# speedup vs baseline: 1.0644x; 1.0644x over previous
import functools

import jax
import jax.numpy as jnp
from jax import lax
from jax.experimental import pallas as pl
from jax.experimental.pallas import tpu as pltpu
from jax.experimental.pallas import tpu_sc as plsc

MAX_LEN = 128
SIDE = MAX_LEN + 1
FLAT = SIDE * SIDE
FLAT_PAD = 16704

_info = plsc.get_sparse_core_info()
NC = _info.num_cores
NS = _info.num_subcores
L = _info.num_lanes
NW = NC * NS


def _make_lookup(B: int):
    b_per_w = B // NW
    n_vec = b_per_w // L
    mesh = plsc.VectorSubcoreMesh(core_axis_name="c", subcore_axis_name="s")

    @functools.partial(
        pl.kernel,
        mesh=mesh,
        out_type=jax.ShapeDtypeStruct((B,), jnp.float32),
        compiler_params=pltpu.CompilerParams(needs_layout_passes=False),
        scratch_types=[
            pltpu.VMEM((FLAT_PAD,), jnp.float32),
            pltpu.VMEM((b_per_w,), jnp.int32),
            pltpu.VMEM((b_per_w,), jnp.int32),
            pltpu.VMEM((b_per_w,), jnp.float32),
            pltpu.SemaphoreType.DMA,
            pltpu.SemaphoreType.DMA,
        ],
    )
    def lookup(
        var_hbm, pre_hbm, tab_hbm, out_hbm, tab_v, var_v, pre_v, out_v, sem_t, sem_i
    ):
        wid = lax.axis_index("s") * NC + lax.axis_index("c")
        base = wid * b_per_w
        cp_t = pltpu.make_async_copy(tab_hbm, tab_v, sem_t)
        cp_v = pltpu.make_async_copy(var_hbm.at[pl.ds(base, b_per_w)], var_v, sem_i)
        cp_p = pltpu.make_async_copy(pre_hbm.at[pl.ds(base, b_per_w)], pre_v, sem_i)
        cp_t.start()
        cp_v.start()
        cp_p.start()
        cp_v.wait()
        cp_p.wait()
        for i in range(n_vec):
            off = i * L
            v = var_v[pl.ds(off, L)]
            p = pre_v[pl.ds(off, L)]
            var_v[pl.ds(off, L)] = v * SIDE + p
        cp_t.wait()
        for i in range(n_vec):
            off = i * L
            out_v[pl.ds(off, L)] = plsc.load_gather(tab_v, [var_v[pl.ds(off, L)]])
        pltpu.sync_copy(out_v, out_hbm.at[pl.ds(base, b_per_w)])

    return lookup


def kernel(var_len, prefix_idx, table):
    B = var_len.shape[0]
    tflat = jnp.pad(table.reshape(-1), (0, FLAT_PAD - FLAT))
    fn = _make_lookup(B)
    return fn(
        var_len.astype(jnp.int32),
        prefix_idx.astype(jnp.int32),
        tflat.astype(jnp.float32),
    )
